# Initial kernel scaffold; baseline (speedup 1.0000x reference)
import jax
import jax.numpy as jnp
from jax.experimental import pallas as pl


def kernel(x, edge_index, batch, emb, Wl, bl, gamma, beta, W1, b1, W2, b2):
    raise NotImplementedError("write your pallas kernel here")


# speedup vs baseline: 8.0179x; 8.0179x over previous
import functools

import jax
import jax.numpy as jnp
from jax import lax
from jax.experimental import pallas as pl
from jax.experimental.pallas import tpu as pltpu
from jax.experimental.pallas import tpu_sc as plsc

N = 10000
E = 160000
H = 256
HH = 128
G = 64
V = 28
L = 4

NC = 2
NS = 16
NW = NC * NS

DEG_EPT = 5008
DEG_PAD = NW * DEG_EPT - E
NH = N + 16

AGG_EPT = E // NS
CH = 125
NCHUNK = AGG_EPT // CH
SLAB = 624
TAIL0 = NS * SLAB
TAILN = N - TAIL0

_mesh = plsc.VectorSubcoreMesh(core_axis_name="c", subcore_axis_name="s")
_sc_params = pltpu.CompilerParams(needs_layout_passes=False)


@functools.partial(
    pl.kernel,
    mesh=_mesh,
    out_type=jax.ShapeDtypeStruct((NW, NH), jnp.float32),
    scratch_types=[
        pltpu.VMEM((DEG_EPT,), jnp.int32),
        pltpu.VMEM((NH,), jnp.float32),
    ],
    compiler_params=_sc_params,
)
def _deg_kernel(col_hbm, zeros_hbm, out_hbm, colv, hist):
    c = lax.axis_index("c")
    s = lax.axis_index("s")
    w = s * NC + c
    pltpu.sync_copy(zeros_hbm, hist)
    pltpu.sync_copy(col_hbm.at[w], colv)
    ones = jnp.full((16,), 1.0, jnp.float32)

    def body(i, carry):
        idx = colv[pl.ds(i * 16, 16)]
        plsc.addupdate_scatter(hist, [idx], ones)
        return carry

    lax.fori_loop(0, DEG_EPT // 16, body, 0)
    pltpu.sync_copy(hist, out_hbm.at[w])


@functools.partial(
    pl.kernel,
    mesh=_mesh,
    out_type=jax.ShapeDtypeStruct((NC, N, HH), jnp.float32),
    scratch_types=[
        pltpu.VMEM((NCHUNK, CH), jnp.int32),
        pltpu.VMEM((NCHUNK, CH), jnp.int32),
        pltpu.VMEM((CH, HH), jnp.float32),
        pltpu.VMEM_SHARED((N, HH), jnp.float32),
    ],
    compiler_params=_sc_params,
)
def _agg_kernel(row_hbm, col_hbm, t_hbm, zeros_hbm, out_hbm, rowv, colv, gbuf, acc):
    c = lax.axis_index("c")
    s = lax.axis_index("s")
    r0 = s * SLAB
    pltpu.sync_copy(zeros_hbm.at[pl.ds(r0, SLAB)], acc.at[pl.ds(r0, SLAB)])

    @pl.when(s == NS - 1)
    def _():
        pltpu.sync_copy(zeros_hbm.at[pl.ds(TAIL0, TAILN)],
                        acc.at[pl.ds(TAIL0, TAILN)])

    pltpu.sync_copy(row_hbm.at[c, s], rowv)
    pltpu.sync_copy(col_hbm.at[s], colv)
    plsc.subcore_barrier()

    def body(j, carry):
        pltpu.sync_copy(t_hbm.at[rowv.at[j]], gbuf)
        pltpu.sync_copy(gbuf, acc.at[colv.at[j]], add=True)
        return carry

    lax.fori_loop(0, NCHUNK, body, 0)
    plsc.subcore_barrier()
    pltpu.sync_copy(acc.at[pl.ds(r0, SLAB)], out_hbm.at[c, pl.ds(r0, SLAB)])

    @pl.when(s == NS - 1)
    def _():
        pltpu.sync_copy(acc.at[pl.ds(TAIL0, TAILN)],
                        out_hbm.at[c, pl.ds(TAIL0, TAILN)])


_NB = 2000


def _prologue_body(x_ref, emb_ref, degt_ref, w0_ref, h_ref, dinv_ref, t2_ref):
    xb = x_ref[...]
    oh = (xb == lax.broadcasted_iota(jnp.int32, (1, V), 1)).astype(jnp.float32)
    h0 = jnp.dot(oh, emb_ref[...], preferred_element_type=jnp.float32)
    deg = jnp.sum(degt_ref[...], axis=1, keepdims=True)
    dinv = jnp.where(deg > 0.0, lax.rsqrt(deg), 0.0)
    tt = dinv * lax.dot_general(h0, w0_ref[...], (((1,), (1,)), ((), ())),
                                preferred_element_type=jnp.float32)
    h_ref[...] = h0
    dinv_ref[...] = dinv
    t2_ref[0] = tt[:, :HH]
    t2_ref[1] = tt[:, HH:]


_prologue = pl.pallas_call(
    _prologue_body,
    grid=(N // _NB,),
    in_specs=[
        pl.BlockSpec((_NB, 1), lambda i: (i, 0)),
        pl.BlockSpec((V, H), lambda i: (0, 0)),
        pl.BlockSpec((_NB, NW), lambda i: (i, 0)),
        pl.BlockSpec((H, H), lambda i: (0, 0)),
    ],
    out_specs=[
        pl.BlockSpec((_NB, H), lambda i: (i, 0)),
        pl.BlockSpec((_NB, 1), lambda i: (i, 0)),
        pl.BlockSpec((NC, _NB, HH), lambda i: (0, i, 0)),
    ],
    out_shape=[
        jax.ShapeDtypeStruct((N, H), jnp.float32),
        jax.ShapeDtypeStruct((N, 1), jnp.float32),
        jax.ShapeDtypeStruct((NC, N, HH), jnp.float32),
    ],
)


def _stats_body(agg_ref, dinv_ref, b_ref, out_ref):
    i = pl.program_id(0)
    ob = jnp.concatenate([agg_ref[0], agg_ref[1]], axis=1)
    ob = dinv_ref[...] * ob + b_ref[...]

    @pl.when(i == 0)
    def _():
        out_ref[...] = jnp.zeros_like(out_ref)

    out_ref[0:1, :] += jnp.sum(ob, axis=0, keepdims=True)
    out_ref[1:2, :] += jnp.sum(ob * ob, axis=0, keepdims=True)


_stats = pl.pallas_call(
    _stats_body,
    grid=(N // _NB,),
    in_specs=[
        pl.BlockSpec((NC, _NB, HH), lambda i: (0, i, 0)),
        pl.BlockSpec((_NB, 1), lambda i: (i, 0)),
        pl.BlockSpec((1, H), lambda i: (0, 0)),
    ],
    out_specs=pl.BlockSpec((2, H), lambda i: (0, 0)),
    out_shape=jax.ShapeDtypeStruct((2, H), jnp.float32),
)


def _apply_body(agg_ref, dinv_ref, b_ref, st_ref, g_ref, be_ref, hp_ref, w_ref,
                h_ref, t2_ref, *, last):
    ob = jnp.concatenate([agg_ref[0], agg_ref[1]], axis=1)
    ob = dinv_ref[...] * ob + b_ref[...]
    mu = st_ref[0:1, :] * (1.0 / N)
    var = st_ref[1:2, :] * (1.0 / N) - mu * mu
    xhat = (ob - mu) * lax.rsqrt(var + 1e-5)
    hn = jnp.maximum(g_ref[...] * xhat + be_ref[...], 0.0) + hp_ref[...]
    h_ref[...] = hn
    if not last:
        tt = dinv_ref[...] * lax.dot_general(
            hn, w_ref[...], (((1,), (1,)), ((), ())),
            preferred_element_type=jnp.float32)
        t2_ref[0] = tt[:, :HH]
        t2_ref[1] = tt[:, HH:]


def _make_apply(last):
    out_specs = [pl.BlockSpec((_NB, H), lambda i: (i, 0))]
    out_shape = [jax.ShapeDtypeStruct((N, H), jnp.float32)]
    if not last:
        out_specs.append(pl.BlockSpec((NC, _NB, HH), lambda i: (0, i, 0)))
        out_shape.append(jax.ShapeDtypeStruct((NC, N, HH), jnp.float32))
    if last:
        def body(agg_ref, dinv_ref, b_ref, st_ref, g_ref, be_ref, hp_ref,
                 w_ref, h_ref):
            _apply_body(agg_ref, dinv_ref, b_ref, st_ref, g_ref, be_ref,
                        hp_ref, w_ref, h_ref, None, last=True)
    else:
        body = functools.partial(_apply_body, last=False)
    return pl.pallas_call(
        body,
        grid=(N // _NB,),
        in_specs=[
            pl.BlockSpec((NC, _NB, HH), lambda i: (0, i, 0)),
            pl.BlockSpec((_NB, 1), lambda i: (i, 0)),
            pl.BlockSpec((1, H), lambda i: (0, 0)),
            pl.BlockSpec((2, H), lambda i: (0, 0)),
            pl.BlockSpec((1, H), lambda i: (0, 0)),
            pl.BlockSpec((1, H), lambda i: (0, 0)),
            pl.BlockSpec((_NB, H), lambda i: (i, 0)),
            pl.BlockSpec((H, H), lambda i: (0, 0)),
        ],
        out_specs=out_specs,
        out_shape=out_shape,
    )


_apply_mid = _make_apply(False)
_apply_last = _make_apply(True)


def _epilogue_body(h_ref, batch_ref, w1_ref, b1_ref, w2_ref, b2_ref, out_ref):
    bb = batch_ref[...]
    oh = (bb == lax.broadcasted_iota(jnp.int32, (1, G), 1)).astype(jnp.float32)
    psum = lax.dot_general(oh, h_ref[...], (((0,), (0,)), ((), ())),
                           preferred_element_type=jnp.float32)
    cnt = lax.dot_general(oh, jnp.ones((N, 1), jnp.float32),
                          (((0,), (0,)), ((), ())),
                          preferred_element_type=jnp.float32)
    pooled = psum / jnp.maximum(cnt, 1.0)
    hid = jnp.maximum(
        lax.dot_general(pooled, w1_ref[...], (((1,), (1,)), ((), ())),
                        preferred_element_type=jnp.float32) + b1_ref[...], 0.0)
    out_ref[...] = (jnp.sum(hid * w2_ref[...], axis=1, keepdims=True)
                    + b2_ref[0, 0])


_epilogue = pl.pallas_call(
    _epilogue_body,
    out_shape=jax.ShapeDtypeStruct((G, 1), jnp.float32),
)


def kernel(x, edge_index, batch, emb, Wl, bl, gamma, beta, W1, b1, W2, b2):
    row = edge_index[0].astype(jnp.int32)
    col = edge_index[1].astype(jnp.int32)

    colp = jnp.concatenate([col, jnp.full((DEG_PAD,), N, jnp.int32)])
    colp = colp.reshape(NW, DEG_EPT)
    deg_parts = _deg_kernel(colp, jnp.zeros((NH,), jnp.float32))
    degt = deg_parts[:, :N].T

    row16 = row.reshape(NS, AGG_EPT)
    row2 = jnp.stack([row16, row16 + N]).reshape(NC, NS, NCHUNK, CH)
    col3 = col.reshape(NS, NCHUNK, CH)
    zeros_acc = jnp.zeros((N, HH), jnp.float32)

    h, dinv, t2 = _prologue(x.reshape(N, 1).astype(jnp.int32), emb, degt, Wl[0])
    for l in range(L):
        t2flat = t2.reshape(NC * N, HH)
        agg = _agg_kernel(row2, col3, t2flat, zeros_acc)
        bvec = bl[l].reshape(1, H)
        gvec = gamma[l].reshape(1, H)
        bevec = beta[l].reshape(1, H)
        st = _stats(agg, dinv, bvec)
        if l < L - 1:
            h, t2 = _apply_mid(agg, dinv, bvec, st, gvec, bevec, h, Wl[l + 1])
        else:
            (h,) = _apply_last(agg, dinv, bvec, st, gvec, bevec, h, Wl[l])

    out = _epilogue(h, batch.reshape(N, 1).astype(jnp.int32), W1,
                    b1.reshape(1, H), W2, b2.reshape(1, 1))
    return (out, jnp.zeros((1,), jnp.float32))
